# baseline (speedup 1.0000x reference)
import functools

import jax
import jax.numpy as jnp
from jax import lax
from jax.experimental import pallas as pl
from jax.experimental.pallas import tpu as pltpu
from jax.experimental.pallas import tpu_sc as plsc

_NC = 2
_NS = 16
_NW = _NC * _NS
_L = 16

_B = 1024
_N = 200
_D = 64
_V = 1000000

_OWN = 32768
_SP = 256
_NBK = 16
_BKT = _OWN // _NBK
_RPB = _BKT // _SP
_PAIR_CAP = 7488
_SLOT = 640
_SORT_CAP = _NBK * _SLOT
_RND_CAP = 832
_FLUSH = 96
_TAIL0 = 33 * _SP
_DUMP = _B * _N


@jax.jit
def _embed_sc(xT, tokT, posT):
    mesh = plsc.VectorSubcoreMesh(core_axis_name="c", subcore_axis_name="s")

    @functools.partial(
        pl.kernel,
        mesh=mesh,
        compiler_params=pltpu.CompilerParams(
            use_tc_tiling_on_sc=True, needs_layout_passes=False
        ),
        out_type=jax.ShapeDtypeStruct((_B * _N + 8, 128), jnp.float32),
        scratch_types=[
            pltpu.VMEM((8, _B // 2), jnp.int32),
            pltpu.VMEM((_PAIR_CAP,), jnp.int32),
            pltpu.VMEM((_PAIR_CAP,), jnp.int32),
            pltpu.VMEM((_SORT_CAP,), jnp.int32),
            pltpu.VMEM((_SORT_CAP,), jnp.int32),
            pltpu.VMEM((_RND_CAP,), jnp.int32),
            pltpu.VMEM((_RND_CAP,), jnp.int32),
            pltpu.VMEM((2, _FLUSH), jnp.int32),
            pltpu.VMEM((2, _D, _SP), jnp.float32),
            pltpu.VMEM((_D, _D), jnp.float32),
            pltpu.VMEM((_D, _N), jnp.float32),
            pltpu.VMEM((2, _FLUSH, 128), jnp.float32),
            pltpu.SemaphoreType.DMA((2,)),
            pltpu.SemaphoreType.DMA((2,)),
        ],
    )
    def k(x_hbm, tok_hbm, pos_hbm, out_hbm, xv, pv, pq, sv, sq, rv, rq,
          rq2d, pbuf, tailv, posv, ov, psem, osem):
        w = lax.axis_index("s") * _NC + lax.axis_index("c")
        lo_w = jnp.where(w < 30, w * _OWN, 983040 + (w - 30) * 8448)
        hi_w = jnp.where(w < 30, lo_w + _OWN, jnp.where(w == 30, 991488, _V))
        nrounds = jnp.where(w < 30, _NBK * _RPB, 33)
        rounds_end = jnp.where(w < 30, _OWN, _TAIL0)
        iota = lax.iota(jnp.int32, _L)
        iota200 = iota * _N
        zero16 = jnp.zeros((_L,), jnp.int32)
        sent_q = jnp.full((_L,), _DUMP, jnp.int32) + (w % 8)

        def extract(src_v, src_q, b0, nq, lo, hi, colbase, dst_v, dst_q, base):
            def body(i, rp):
                for u in range(4):
                    sl = pl.ds(b0 + i * 64 + u * _L, _L)
                    vl = src_v[sl]
                    q = src_q[sl]
                    m = (vl >= lo) & (vl < hi)
                    rk = plsc.cumsum(m.astype(jnp.int32))
                    slot = rp + rk - 1
                    plsc.store_scatter(dst_v, [slot], vl - colbase, mask=m)
                    plsc.store_scatter(dst_q, [slot], q, mask=m)
                    rp = rp + plsc.all_reduce_population_count(m)
                return rp

            return lax.fori_loop(0, nq, body, base)

        def pad4(dst_v, dst_q, end, qfill):
            for u in range(4):
                plsc.store_scatter(dst_v, [end + u * _L + iota], zero16)
                plsc.store_scatter(dst_q, [end + u * _L + iota], qfill)

        pltpu.sync_copy(pos_hbm, posv)
        pltpu.sync_copy(tok_hbm.at[:, pl.ds(_V - _D, _D)], tailv)

        def scan_block(rbh, ptr):
            rb = rbh // 2
            half = rbh % 2
            pltpu.sync_copy(
                x_hbm.at[pl.ds(rb * 8, 8), pl.ds(half * (_B // 2), _B // 2)], xv
            )

            def cc_body(cc, p):
                for r in range(8):
                    v = xv[r, pl.ds(cc * _L, _L)]
                    q = iota200 + ((half * (_B // 2) + cc * _L) * _N
                                   + rb * 8 + r)
                    m = (v >= lo_w) & (v < hi_w)
                    rk = plsc.cumsum(m.astype(jnp.int32))
                    slot = p + rk - 1
                    plsc.store_scatter(pv, [slot], v - lo_w, mask=m)
                    plsc.store_scatter(pq, [slot], q, mask=m)
                    p = p + plsc.all_reduce_population_count(m)
                return p

            return lax.fori_loop(0, _B // 2 // _L, cc_body, ptr)

        def prefill(i, _):
            for u in range(4):
                plsc.store_scatter(sv, [i * 64 + u * _L + iota], zero16)
                plsc.store_scatter(sq, [i * 64 + u * _L + iota], sent_q)
            return 0

        lax.fori_loop(0, _SORT_CAP // 64, prefill, 0)

        ptr = lax.fori_loop(0, _N // 4, scan_block, zero16)
        pad4(pv, pq, ptr, sent_q)
        nq_all = (jnp.max(ptr) + 63) // 64

        def bucket_body(bk, _):
            extract(pv, pq, 0, nq_all, bk * _BKT, (bk + 1) * _BKT, 0,
                    sv, sq, jnp.full((_L,), bk * _SLOT, jnp.int32))
            return 0

        lax.fori_loop(0, _NBK, bucket_body, 0)

        def start_panel(g, half):
            pltpu.make_async_copy(
                tok_hbm.at[:, pl.ds(lo_w + g * _SP, _SP)],
                pbuf.at[half],
                psem.at[half],
            ).start()

        def wait_panel(half):
            pltpu.make_async_copy(
                tok_hbm.at[:, pl.ds(0, _SP)], pbuf.at[half], psem.at[half]
            ).wait()

        def flush_rounds(src_ref, rptr, ub):
            pad4(rv, rq, rptr, sent_q)
            pad4(rv, rq, rptr + 64, sent_q)
            nfl = (jnp.max(rptr) + _FLUSH - 1) // _FLUSH

            def one_flush(f, i, ubi):
                live = f < nfl

                @pl.when(live & (ubi == 1))
                def _():
                    pltpu.make_async_copy(
                        out_hbm.at[pl.ds(0, _FLUSH)], ov.at[i], osem.at[i]
                    ).wait()

                @pl.when(live)
                def _():
                    for g in range(_FLUSH // _L):
                        sl = pl.ds(f * _FLUSH + g * _L, _L)
                        col = rv[sl]
                        q = rq[sl]
                        j = lax.rem(q, _N)
                        slot = iota + g * _L

                        def eblk(eb, _, col=col, j=j, slot=slot):
                            for ee in range(8):
                                ef = eb * 8 + ee
                                efv = jnp.full((_L,), 0, jnp.int32) + ef
                                t = plsc.load_gather(src_ref, [efv, col])
                                p = plsc.load_gather(posv, [efv, j])
                                plsc.store_scatter(ov.at[i], [slot, efv], t + p)
                            return 0

                        lax.fori_loop(0, _D // 8, eblk, 0)
                        rq2d[i, pl.ds(g * _L, _L)] = q
                    pltpu.make_async_copy(
                        ov.at[i], out_hbm.at[rq2d.at[i]], osem.at[i]
                    ).start()

                return jnp.where(live, jnp.int32(1), ubi)

            def fpair(ff, carry):
                return tuple(
                    one_flush(2 * ff + i, i, carry[i]) for i in range(2)
                )

            return lax.fori_loop(0, (nfl + 1) // 2, fpair, ub)

        start_panel(0, 0)
        ub = (jnp.int32(0),) * 2
        snq = _SLOT // 64

        def round_pair(kk, ub):
            for h in range(2):
                g = 2 * kk + h

                @pl.when(g + 1 < nrounds)
                def _():
                    start_panel(g + 1, 1 - h)

                @pl.when(g < nrounds)
                def _():
                    wait_panel(h)

                b0 = (g // _RPB) * _SLOT
                rlo = g * _SP
                rhi = jnp.minimum(rlo + _SP, rounds_end)
                rptr = extract(sv, sq, b0, snq, rlo, rhi, rlo,
                               rv, rq, zero16)
                ub = flush_rounds(pbuf.at[h], rptr, ub)
            return ub

        ub = lax.fori_loop(0, _NBK * _RPB // 2, round_pair, ub)

        hi_tail = _TAIL0 + jnp.where(w == _NW - 1, _D, 0)
        rptr = extract(sv, sq, (_TAIL0 // _BKT) * _SLOT, snq,
                       _TAIL0, hi_tail, _TAIL0, rv, rq, zero16)
        ub = flush_rounds(tailv, rptr, ub)

        for i in range(2):
            @pl.when(ub[i] == 1)
            def _(i=i):
                pltpu.make_async_copy(
                    out_hbm.at[pl.ds(0, _FLUSH)], ov.at[i], osem.at[i]
                ).wait()

    return k(xT, tokT, posT)


def kernel(x, tok_table, pos_table):
    b, n = x.shape
    v, d = tok_table.shape
    out = _embed_sc(x.T, tok_table.T, pos_table.T)
    return out[: b * n, :d].reshape(b, n, d)


# speedup vs baseline: 1.6330x; 1.6330x over previous
import functools

import jax
import jax.numpy as jnp
from jax import lax
from jax.experimental import pallas as pl
from jax.experimental.pallas import tpu as pltpu
from jax.experimental.pallas import tpu_sc as plsc

_NC = 2
_NS = 16
_NW = _NC * _NS
_L = 16

_B = 1024
_N = 200
_D = 64
_V = 1000000

_OWN = 32768
_SP = 256
_NBK = 16
_BKT = _OWN // _NBK
_RPB = _BKT // _SP
_PAIR_CAP = 7488
_SLOT = 640
_SORT_CAP = _NBK * _SLOT
_RND_CAP = 832
_FLUSH = 32
_TAIL0 = 33 * _SP
_DUMP = _B * _N


@jax.jit
def _embed_sc(xT, tokT, posT):
    mesh = plsc.VectorSubcoreMesh(core_axis_name="c", subcore_axis_name="s")

    @functools.partial(
        pl.kernel,
        mesh=mesh,
        compiler_params=pltpu.CompilerParams(
            use_tc_tiling_on_sc=True, needs_layout_passes=False
        ),
        out_type=jax.ShapeDtypeStruct((_B * _N + 8, 128), jnp.float32),
        scratch_types=[
            pltpu.VMEM((8, _B // 2), jnp.int32),
            pltpu.VMEM((_PAIR_CAP,), jnp.int32),
            pltpu.VMEM((_PAIR_CAP,), jnp.int32),
            pltpu.VMEM((_SORT_CAP,), jnp.int32),
            pltpu.VMEM((_SORT_CAP,), jnp.int32),
            pltpu.VMEM((_RND_CAP,), jnp.int32),
            pltpu.VMEM((_RND_CAP,), jnp.int32),
            pltpu.VMEM((2, _FLUSH), jnp.int32),
            pltpu.VMEM((2, _D, _SP), jnp.float32),
            pltpu.VMEM((_D, _D), jnp.float32),
            pltpu.VMEM((_D, _N), jnp.float32),
            pltpu.VMEM((2, _FLUSH, 128), jnp.float32),
            pltpu.SemaphoreType.DMA((2,)),
            pltpu.SemaphoreType.DMA((2,)),
        ],
    )
    def k(x_hbm, tok_hbm, pos_hbm, out_hbm, xv, pv, pq, sv, sq, rv, rq,
          rq2d, pbuf, tailv, posv, ov, psem, osem):
        w = lax.axis_index("s") * _NC + lax.axis_index("c")
        lo_w = jnp.where(w < 30, w * _OWN, 983040 + (w - 30) * 8448)
        hi_w = jnp.where(w < 30, lo_w + _OWN, jnp.where(w == 30, 991488, _V))
        nrounds = jnp.where(w < 30, _NBK * _RPB, 33)
        rounds_end = jnp.where(w < 30, _OWN, _TAIL0)
        iota = lax.iota(jnp.int32, _L)
        iota200 = iota * _N
        zero16 = jnp.zeros((_L,), jnp.int32)
        sent_q = jnp.full((_L,), _DUMP, jnp.int32) + (w % 8)

        def extract(src_v, src_q, b0, nq, lo, hi, colbase, dst_v, dst_q, base):
            def body(i, rp):
                for u in range(4):
                    sl = pl.ds(b0 + i * 64 + u * _L, _L)
                    vl = src_v[sl]
                    q = src_q[sl]
                    m = (vl >= lo) & (vl < hi)
                    rk = plsc.cumsum(m.astype(jnp.int32))
                    slot = rp + rk - 1
                    plsc.store_scatter(dst_v, [slot], vl - colbase, mask=m)
                    plsc.store_scatter(dst_q, [slot], q, mask=m)
                    rp = rp + plsc.all_reduce_population_count(m)
                return rp

            return lax.fori_loop(0, nq, body, base)

        def pad4(dst_v, dst_q, end, qfill):
            for u in range(4):
                plsc.store_scatter(dst_v, [end + u * _L + iota], zero16)
                plsc.store_scatter(dst_q, [end + u * _L + iota], qfill)

        pltpu.sync_copy(pos_hbm, posv)
        pltpu.sync_copy(tok_hbm.at[:, pl.ds(_V - _D, _D)], tailv)

        def scan_block(rbh, ptr):
            rb = rbh // 2
            half = rbh % 2
            pltpu.sync_copy(
                x_hbm.at[pl.ds(rb * 8, 8), pl.ds(half * (_B // 2), _B // 2)], xv
            )

            def cc_body(cc, p):
                for r in range(8):
                    v = xv[r, pl.ds(cc * _L, _L)]
                    q = iota200 + ((half * (_B // 2) + cc * _L) * _N
                                   + rb * 8 + r)
                    m = (v >= lo_w) & (v < hi_w)
                    rk = plsc.cumsum(m.astype(jnp.int32))
                    slot = p + rk - 1
                    plsc.store_scatter(pv, [slot], v - lo_w, mask=m)
                    plsc.store_scatter(pq, [slot], q, mask=m)
                    p = p + plsc.all_reduce_population_count(m)
                return p

            return lax.fori_loop(0, _B // 2 // _L, cc_body, ptr)

        def prefill(i, _):
            for u in range(4):
                plsc.store_scatter(sv, [i * 64 + u * _L + iota], zero16)
                plsc.store_scatter(sq, [i * 64 + u * _L + iota], sent_q)
            return 0

        lax.fori_loop(0, _SORT_CAP // 64, prefill, 0)

        ptr = lax.fori_loop(0, _N // 4, scan_block, zero16)
        pad4(pv, pq, ptr, sent_q)
        nq_all = (jnp.max(ptr) + 63) // 64

        def bucket_body(bk, _):
            extract(pv, pq, 0, nq_all, bk * _BKT, (bk + 1) * _BKT, 0,
                    sv, sq, jnp.full((_L,), bk * _SLOT, jnp.int32))
            return 0

        lax.fori_loop(0, _NBK, bucket_body, 0)

        def start_panel(g, half):
            pltpu.make_async_copy(
                tok_hbm.at[:, pl.ds(lo_w + g * _SP, _SP)],
                pbuf.at[half],
                psem.at[half],
            ).start()

        def wait_panel(half):
            pltpu.make_async_copy(
                tok_hbm.at[:, pl.ds(0, _SP)], pbuf.at[half], psem.at[half]
            ).wait()

        def flush_rounds(src_ref, rptr, ub):
            pad4(rv, rq, rptr, sent_q)
            pad4(rv, rq, rptr + 64, sent_q)
            nfl = (jnp.max(rptr) + _FLUSH - 1) // _FLUSH

            def one_flush(f, i, ubi):
                live = f < nfl

                @pl.when(live & (ubi == 1))
                def _():
                    pltpu.make_async_copy(
                        out_hbm.at[pl.ds(0, _FLUSH)], ov.at[i], osem.at[i]
                    ).wait()

                @pl.when(live)
                def _():
                    for g in range(_FLUSH // _L):
                        sl = pl.ds(f * _FLUSH + g * _L, _L)
                        col = rv[sl]
                        q = rq[sl]
                        j = lax.rem(q, _N)
                        slot = iota + g * _L

                        def eblk(eb, _, col=col, j=j, slot=slot):
                            for ee in range(8):
                                ef = eb * 8 + ee
                                efv = jnp.full((_L,), 0, jnp.int32) + ef
                                t = plsc.load_gather(src_ref, [efv, col])
                                p = plsc.load_gather(posv, [efv, j])
                                plsc.store_scatter(ov.at[i], [slot, efv], t + p)
                            return 0

                        lax.fori_loop(0, _D // 8, eblk, 0)
                        rq2d[i, pl.ds(g * _L, _L)] = q
                    pltpu.make_async_copy(
                        ov.at[i], out_hbm.at[rq2d.at[i]], osem.at[i]
                    ).start()

                return jnp.where(live, jnp.int32(1), ubi)

            def fpair(ff, carry):
                return tuple(
                    one_flush(2 * ff + i, i, carry[i]) for i in range(2)
                )

            return lax.fori_loop(0, (nfl + 1) // 2, fpair, ub)

        start_panel(0, 0)
        ub = (jnp.int32(0),) * 2
        snq = _SLOT // 64

        def round_pair(kk, ub):
            for h in range(2):
                g = 2 * kk + h

                @pl.when(g + 1 < nrounds)
                def _():
                    start_panel(g + 1, 1 - h)

                @pl.when(g < nrounds)
                def _():
                    wait_panel(h)

                b0 = (g // _RPB) * _SLOT
                rlo = g * _SP
                rhi = jnp.minimum(rlo + _SP, rounds_end)
                rptr = extract(sv, sq, b0, snq, rlo, rhi, rlo,
                               rv, rq, zero16)
                ub = flush_rounds(pbuf.at[h], rptr, ub)
            return ub

        ub = lax.fori_loop(0, _NBK * _RPB // 2, round_pair, ub)

        hi_tail = _TAIL0 + jnp.where(w == _NW - 1, _D, 0)
        rptr = extract(sv, sq, (_TAIL0 // _BKT) * _SLOT, snq,
                       _TAIL0, hi_tail, _TAIL0, rv, rq, zero16)
        ub = flush_rounds(tailv, rptr, ub)

        for i in range(2):
            @pl.when(ub[i] == 1)
            def _(i=i):
                pltpu.make_async_copy(
                    out_hbm.at[pl.ds(0, _FLUSH)], ov.at[i], osem.at[i]
                ).wait()

    return k(xT, tokT, posT)


def kernel(x, tok_table, pos_table):
    b, n = x.shape
    v, d = tok_table.shape
    out = _embed_sc(x.T, tok_table.T, pos_table.T)
    return out[: b * n, :d].reshape(b, n, d)


# speedup vs baseline: 1.9956x; 1.2220x over previous
import functools

import jax
import jax.numpy as jnp
from jax import lax
from jax.experimental import pallas as pl
from jax.experimental.pallas import tpu as pltpu
from jax.experimental.pallas import tpu_sc as plsc

_NC = 2
_NS = 16
_NW = _NC * _NS
_LANES = 16
_CHUNK = 128
_NBUF = 5

_TC_BLK = 2048


@jax.jit
def _transpose_tc(tokT):
    d, v = tokT.shape
    grid = (v + _TC_BLK - 1) // _TC_BLK

    def body(in_ref, out_ref):
        out_ref[...] = jnp.swapaxes(in_ref[...], 0, 1).reshape(_TC_BLK * d)

    return pl.pallas_call(
        body,
        grid=(grid,),
        in_specs=[pl.BlockSpec((d, _TC_BLK), lambda i: (0, i))],
        out_specs=pl.BlockSpec((_TC_BLK * d,), lambda i: (i,)),
        out_shape=jax.ShapeDtypeStruct((v * d,), jnp.float32),
    )(tokT)


@functools.partial(jax.jit, static_argnames=("b", "n", "d"))
def _embed_sc(x3, tok_table, pos_table, b, n, d):
    total = b * n
    cpw = total // (_NW * _CHUNK)

    mesh = plsc.VectorSubcoreMesh(core_axis_name="c", subcore_axis_name="s")

    @functools.partial(
        pl.kernel,
        mesh=mesh,
        compiler_params=pltpu.CompilerParams(use_tc_tiling_on_sc=False),
        out_type=jax.ShapeDtypeStruct((total, d), jnp.float32),
        scratch_types=[
            pltpu.VMEM((cpw, _CHUNK), jnp.int32),
            pltpu.VMEM((_NBUF, _CHUNK, d), jnp.float32),
            pltpu.VMEM((2 * n, d), jnp.float32),
            pltpu.SemaphoreType.DMA((_NBUF,)),
            pltpu.SemaphoreType.DMA((_NBUF,)),
        ],
    )
    def k(x_hbm, tok_hbm, pos_hbm, out_hbm, idx_v, rows_v, pos_v, gsem, osem):
        wid = lax.axis_index("s") * _NC + lax.axis_index("c")
        pltpu.sync_copy(pos_hbm, pos_v.at[pl.ds(0, n)])
        pltpu.sync_copy(pos_hbm, pos_v.at[pl.ds(n, n)])
        pltpu.sync_copy(x_hbm.at[wid], idx_v)

        def fire_gather(i, b_):
            pltpu.make_async_copy(
                tok_hbm.at[idx_v.at[i]], rows_v.at[b_], gsem.at[b_]
            ).start()

        def out_slice(i):
            g = wid * cpw + i
            return out_hbm.at[pl.ds(g * _CHUNK, _CHUNK)]

        for b_ in range(_NBUF):
            fire_gather(b_, b_)

        @pl.loop(0, cpw, step=_NBUF)
        def _(i0):
            for b_ in range(_NBUF):
                i = i0 + b_
                pltpu.make_async_copy(
                    tok_hbm.at[idx_v.at[0]], rows_v.at[b_], gsem.at[b_]
                ).wait()
                g = wid * cpw + i
                po = (g * _CHUNK) % n

                @pl.loop(0, _CHUNK)
                def _(r):
                    for c in range(d // _LANES):
                        sl = pl.ds(c * _LANES, _LANES)
                        plsc.addupdate(rows_v.at[b_, r, sl], pos_v[po + r, sl])

                pltpu.make_async_copy(rows_v.at[b_], out_slice(i), osem.at[b_]).start()

                @pl.when(i + _NBUF < cpw)
                def _():
                    pltpu.make_async_copy(
                        rows_v.at[b_], out_slice(0), osem.at[b_]
                    ).wait()
                    fire_gather(i + _NBUF, b_)

        for b_ in range(_NBUF):
            pltpu.make_async_copy(rows_v.at[b_], out_slice(0), osem.at[b_]).wait()

    return k(x3, tok_table, pos_table)


def kernel(x, tok_table, pos_table):
    b, n = x.shape
    v, d = tok_table.shape
    cpw = (b * n) // (_NW * _CHUNK)
    x3 = x.reshape(-1).astype(jnp.int32).reshape(_NW, cpw, _CHUNK)
    tok_flat = lax.optimization_barrier(tok_table.reshape(-1))
    out = _embed_sc(x3, tok_flat.reshape(v, d), pos_table, b, n, d)
    return out.reshape(b, n, d)
